# Initial kernel scaffold; baseline (speedup 1.0000x reference)
import jax
import jax.numpy as jnp
from jax.experimental import pallas as pl


def kernel(edge_ids, node_features, link_labels, event_nums, emb, lW1, lb1, lW2, lb2, mW, mb, ulW, ulb, W_ih, W_hh, b_ih, b_hh, rW1, rb1, rW2, rb2):
    raise NotImplementedError("write your pallas kernel here")


# speedup vs baseline: 1.0339x; 1.0339x over previous
import jax
import jax.numpy as jnp
import numpy as np
from jax.experimental import pallas as pl


def _gpnn_body(ids_ref, nf_ref,
               emb_ref, lW1T_ref, lb1_ref, lW2T_ref, lb2_ref,
               mWwT_ref, mWvT_ref, mWeT_ref, mb_ref,
               ulWeT_ref, ulWmT_ref, ulb_ref,
               W_ihT_ref, W_hhT_ref, b_ih_ref, b_hh_ref,
               rW1aT_ref, rW1bT_ref, rb1_ref, rW2T_ref, rb2_ref,
               out_ref):
    N = ids_ref.shape[1]
    P = N * N
    NF = nf_ref.shape[2]
    EF = emb_ref.shape[1]

    ids = ids_ref[0, :, :, 0].reshape(P, 1)

    emb = emb_ref[...]
    e = jnp.zeros((P, EF), jnp.float32)
    for k in range(emb.shape[0]):
        e = e + jnp.where(ids == k, 1.0, 0.0) * emb[k][None, :]

    h = nf_ref[0]

    for _ in range(2):
        x = jnp.maximum(jnp.dot(e, lW1T_ref[...],
                                preferred_element_type=jnp.float32)
                        + lb1_ref[...], 0.0)
        adj = jnp.dot(x, lW2T_ref[...],
                      preferred_element_type=jnp.float32) + lb2_ref[...]
        sgm = jax.nn.sigmoid(adj)

        A = jnp.dot(h, mWwT_ref[...], preferred_element_type=jnp.float32)
        Bv = jnp.dot(h, mWvT_ref[...], preferred_element_type=jnp.float32)
        em = jnp.dot(e, mWeT_ref[...], preferred_element_type=jnp.float32)
        MS = em.shape[1]
        m3 = (em.reshape(N, N, MS) + A[:, None, :] + Bv[None, :, :]
              + mb_ref[...][None, :, :])
        m3 = sgm.reshape(N, N, 1) * m3
        mflat = m3.reshape(P, MS)

        e = (jnp.dot(e, ulWeT_ref[...], preferred_element_type=jnp.float32)
             + jnp.dot(mflat, ulWmT_ref[...],
                       preferred_element_type=jnp.float32)
             + ulb_ref[...])

        msum = jnp.sum(m3, axis=0)
        gi = jnp.dot(msum, W_ihT_ref[...],
                     preferred_element_type=jnp.float32) + b_ih_ref[...]
        gh = jnp.dot(h, W_hhT_ref[...],
                     preferred_element_type=jnp.float32) + b_hh_ref[...]
        r = jax.nn.sigmoid(gi[:, :NF] + gh[:, :NF])
        z = jax.nn.sigmoid(gi[:, NF:2 * NF] + gh[:, NF:2 * NF])
        n = jnp.tanh(gi[:, 2 * NF:] + r * gh[:, 2 * NF:])
        h = (1.0 - z) * n + z * h

    eT = e.reshape(N, N, EF).transpose(1, 0, 2).reshape(P, EF)
    rx = jnp.maximum(
        jnp.dot(e, rW1aT_ref[...], preferred_element_type=jnp.float32)
        + jnp.dot(eT, rW1bT_ref[...], preferred_element_type=jnp.float32)
        + rb1_ref[...], 0.0)
    out_ref[0] = (jnp.dot(rx, rW2T_ref[...],
                          preferred_element_type=jnp.float32)
                  + rb2_ref[...])


def kernel(edge_ids, node_features, link_labels, event_nums, emb, lW1, lb1,
           lW2, lb2, mW, mb, ulW, ulb, W_ih, W_hh, b_ih, b_hh, rW1, rb1,
           rW2, rb2):
    B, N, _, _ = edge_ids.shape
    NF = node_features.shape[2]
    EF = emb.shape[1]
    P = N * N

    lW1T = lW1.T
    lW2T = lW2.T
    mWwT = mW[:, :NF].T
    mWvT = mW[:, NF:2 * NF].T
    mWeT = mW[:, 2 * NF:].T
    ulWeT = ulW[:, :EF].T
    ulWmT = ulW[:, EF:].T
    W_ihT = W_ih.T
    W_hhT = W_hh.T
    rW1aT = rW1[:, :EF].T
    rW1bT = rW1[:, EF:].T
    rW2T = jnp.pad(rW2.T, ((0, 0), (0, 6)))
    rb2p = jnp.pad(rb2, (0, 6))

    r2 = lambda a: a.reshape(1, -1)

    full = lambda shape: pl.BlockSpec(shape, lambda b: (0,) * len(shape))
    in_specs = [
        pl.BlockSpec((1, N, N, 1), lambda b: (b, 0, 0, 0)),
        pl.BlockSpec((1, N, NF), lambda b: (b, 0, 0)),
        full(emb.shape), full(lW1T.shape), full((1, lb1.shape[0])),
        full(lW2T.shape), full((1, 1)),
        full(mWwT.shape), full(mWvT.shape), full(mWeT.shape),
        full((1, mb.shape[0])),
        full(ulWeT.shape), full(ulWmT.shape), full((1, ulb.shape[0])),
        full(W_ihT.shape), full(W_hhT.shape),
        full((1, b_ih.shape[0])), full((1, b_hh.shape[0])),
        full(rW1aT.shape), full(rW1bT.shape), full((1, rb1.shape[0])),
        full(rW2T.shape), full((1, rb2p.shape[0])),
    ]

    ro = pl.pallas_call(
        _gpnn_body,
        grid=(B,),
        in_specs=in_specs,
        out_specs=pl.BlockSpec((1, P, 16), lambda b: (b, 0, 0)),
        out_shape=jax.ShapeDtypeStruct((B, P, 16), jnp.float32),
    )(edge_ids, node_features,
      emb, lW1T, r2(lb1), lW2T, r2(lb2),
      mWwT, mWvT, mWeT, r2(mb),
      ulWeT, ulWmT, r2(ulb),
      W_ihT, W_hhT, r2(b_ih), r2(b_hh),
      rW1aT, rW1bT, r2(rb1), rW2T, r2(rb2p))

    iu, ju = np.triu_indices(N, k=1)
    L = iu.shape[0]
    tri = ro.reshape(B, N, N, 16)[:, iu, ju, :10]
    return tri.reshape(B, L, 5, 2).transpose(0, 2, 1, 3)


# speedup vs baseline: 1.1087x; 1.0724x over previous
import jax
import jax.numpy as jnp
import numpy as np
from jax.experimental import pallas as pl


def _gpnn_body(ids_ref, nf_ref,
               emb_ref, lW1T_ref, lb1_ref, lW2T_ref, lb2_ref,
               mWwT_ref, mWvT_ref, mWeT_ref, mb_ref,
               ulWeT_ref, ulWmT_ref, ulb_ref,
               W_ihT_ref, W_hhT_ref, b_ih_ref, b_hh_ref,
               rW1aT_ref, rW1bT_ref, rb1_ref, rW2T_ref, rb2_ref,
               out_ref):
    N = ids_ref.shape[1]
    P = N * N
    NF = nf_ref.shape[2]
    EF = emb_ref.shape[1]

    f32 = jnp.float32
    ids = ids_ref[0]

    emb = emb_ref[...]
    x1t = jnp.maximum(jnp.dot(emb, lW1T_ref[...],
                              preferred_element_type=f32) + lb1_ref[...], 0.0)
    s1t = jax.nn.sigmoid(jnp.dot(x1t, lW2T_ref[...],
                                 preferred_element_type=f32) + lb2_ref[...])
    tmt = jnp.dot(emb, mWeT_ref[...], preferred_element_type=f32) + mb_ref[...]
    tgt = s1t * tmt
    T2t = (jnp.dot(emb, ulWeT_ref[...], preferred_element_type=f32)
           + ulb_ref[...]
           + s1t * jnp.dot(tmt, ulWmT_ref[...], preferred_element_type=f32))

    oh = [(ids == k).astype(f32) for k in range(4)]
    S = (oh[0] * s1t[0, 0] + oh[1] * s1t[1, 0]
         + oh[2] * s1t[2, 0] + oh[3] * s1t[3, 0])

    h = nf_ref[0]

    A1 = jnp.dot(h, mWwT_ref[...], preferred_element_type=f32)
    B1 = jnp.dot(h, mWvT_ref[...], preferred_element_type=f32)
    UA = jnp.dot(A1, ulWmT_ref[...], preferred_element_type=f32)
    UB = jnp.dot(B1, ulWmT_ref[...], preferred_element_type=f32)

    cnt = jnp.stack([jnp.sum(o, axis=0) for o in oh], axis=1)
    s0 = jnp.sum(S, axis=0)
    msum1 = (jnp.dot(cnt, tgt, preferred_element_type=f32)
             + jnp.dot(S.T, A1, preferred_element_type=f32)
             + s0[:, None] * B1)

    gi = jnp.dot(msum1, W_ihT_ref[...], preferred_element_type=f32) + b_ih_ref[...]
    gh = jnp.dot(h, W_hhT_ref[...], preferred_element_type=f32) + b_hh_ref[...]
    r = jax.nn.sigmoid(gi[:, :NF] + gh[:, :NF])
    z = jax.nn.sigmoid(gi[:, NF:2 * NF] + gh[:, NF:2 * NF])
    n = jnp.tanh(gi[:, 2 * NF:] + r * gh[:, 2 * NF:])
    h = (1.0 - z) * n + z * h

    t2g = (oh[0][:, :, None] * T2t[0] + oh[1][:, :, None] * T2t[1]
           + oh[2][:, :, None] * T2t[2] + oh[3][:, :, None] * T2t[3])
    e2 = (t2g + S[:, :, None] * (UA[:, None, :] + UB[None, :, :])
          ).reshape(P, EF)

    x2 = jnp.maximum(jnp.dot(e2, lW1T_ref[...],
                             preferred_element_type=f32) + lb1_ref[...], 0.0)
    sgm2 = jax.nn.sigmoid(jnp.dot(x2, lW2T_ref[...],
                                  preferred_element_type=f32) + lb2_ref[...])
    A2 = jnp.dot(h, mWwT_ref[...], preferred_element_type=f32)
    B2 = jnp.dot(h, mWvT_ref[...], preferred_element_type=f32)
    em2 = jnp.dot(e2, mWeT_ref[...], preferred_element_type=f32)
    MS = em2.shape[1]
    m2 = (sgm2.reshape(N, N, 1)
          * (em2.reshape(N, N, MS) + A2[:, None, :] + B2[None, :, :]
             + mb_ref[...][None, :, :])).reshape(P, MS)
    e3 = (jnp.dot(e2, ulWeT_ref[...], preferred_element_type=f32)
          + jnp.dot(m2, ulWmT_ref[...], preferred_element_type=f32)
          + ulb_ref[...])

    e3T = e3.reshape(N, N, EF).transpose(1, 0, 2).reshape(P, EF)
    rx = jnp.maximum(
        jnp.dot(e3, rW1aT_ref[...], preferred_element_type=f32)
        + jnp.dot(e3T, rW1bT_ref[...], preferred_element_type=f32)
        + rb1_ref[...], 0.0)
    out_ref[0] = (jnp.dot(rx, rW2T_ref[...], preferred_element_type=f32)
                  + rb2_ref[...])


def kernel(edge_ids, node_features, link_labels, event_nums, emb, lW1, lb1,
           lW2, lb2, mW, mb, ulW, ulb, W_ih, W_hh, b_ih, b_hh, rW1, rb1,
           rW2, rb2):
    B, N, _, _ = edge_ids.shape
    NF = node_features.shape[2]
    EF = emb.shape[1]
    P = N * N

    lW1T = lW1.T
    lW2T = lW2.T
    mWwT = mW[:, :NF].T
    mWvT = mW[:, NF:2 * NF].T
    mWeT = mW[:, 2 * NF:].T
    ulWeT = ulW[:, :EF].T
    ulWmT = ulW[:, EF:].T
    W_ihT = W_ih.T
    W_hhT = W_hh.T
    rW1aT = rW1[:, :EF].T
    rW1bT = rW1[:, EF:].T
    rW2T = jnp.pad(rW2.T, ((0, 0), (0, 6)))
    rb2p = jnp.pad(rb2, (0, 6))

    r2 = lambda a: a.reshape(1, -1)

    full = lambda shape: pl.BlockSpec(shape, lambda b: (0,) * len(shape))
    in_specs = [
        pl.BlockSpec((1, N, N), lambda b: (b, 0, 0)),
        pl.BlockSpec((1, N, NF), lambda b: (b, 0, 0)),
        full(emb.shape), full(lW1T.shape), full((1, lb1.shape[0])),
        full(lW2T.shape), full((1, 1)),
        full(mWwT.shape), full(mWvT.shape), full(mWeT.shape),
        full((1, mb.shape[0])),
        full(ulWeT.shape), full(ulWmT.shape), full((1, ulb.shape[0])),
        full(W_ihT.shape), full(W_hhT.shape),
        full((1, b_ih.shape[0])), full((1, b_hh.shape[0])),
        full(rW1aT.shape), full(rW1bT.shape), full((1, rb1.shape[0])),
        full(rW2T.shape), full((1, rb2p.shape[0])),
    ]

    ro = pl.pallas_call(
        _gpnn_body,
        grid=(B,),
        in_specs=in_specs,
        out_specs=pl.BlockSpec((1, P, 16), lambda b: (b, 0, 0)),
        out_shape=jax.ShapeDtypeStruct((B, P, 16), jnp.float32),
    )(edge_ids.reshape(B, N, N), node_features,
      emb, lW1T, r2(lb1), lW2T, r2(lb2),
      mWwT, mWvT, mWeT, r2(mb),
      ulWeT, ulWmT, r2(ulb),
      W_ihT, W_hhT, r2(b_ih), r2(b_hh),
      rW1aT, rW1bT, r2(rb1), rW2T, r2(rb2p))

    iu, ju = np.triu_indices(N, k=1)
    L = iu.shape[0]
    tri = ro.reshape(B, N, N, 16)[:, iu, ju, :10]
    return tri.reshape(B, L, 5, 2).transpose(0, 2, 1, 3)


# speedup vs baseline: 1.9935x; 1.7981x over previous
import jax
import jax.numpy as jnp
import numpy as np
from jax.experimental import pallas as pl


def _gpnn_body(ids_ref, nf_ref,
               emb_ref, lW1T_ref, lb1_ref, lW2T_ref, lb2_ref,
               mWwT_ref, mWvT_ref, mWeT_ref, mb_ref,
               ulWeT_ref, ulWmT_ref, ulb_ref,
               W_ihT_ref, W_hhT_ref, b_ih_ref, b_hh_ref,
               rW1aT_ref, rW1bT_ref, rb1_ref, rW2T_ref, rb2_ref,
               out_ref):
    N = ids_ref.shape[1]
    P = N * N
    NF = nf_ref.shape[2]
    EF = emb_ref.shape[1]

    f32 = jnp.float32
    ids = ids_ref[0]

    emb = emb_ref[...]
    x1t = jnp.maximum(jnp.dot(emb, lW1T_ref[...],
                              preferred_element_type=f32) + lb1_ref[...], 0.0)
    s1t = jax.nn.sigmoid(jnp.dot(x1t, lW2T_ref[...],
                                 preferred_element_type=f32) + lb2_ref[...])
    tmt = jnp.dot(emb, mWeT_ref[...], preferred_element_type=f32) + mb_ref[...]
    tgt = s1t * tmt
    T2t = (jnp.dot(emb, ulWeT_ref[...], preferred_element_type=f32)
           + ulb_ref[...]
           + s1t * jnp.dot(tmt, ulWmT_ref[...], preferred_element_type=f32))

    oh = [(ids == k).astype(f32) for k in range(4)]
    S = (oh[0] * s1t[0, 0] + oh[1] * s1t[1, 0]
         + oh[2] * s1t[2, 0] + oh[3] * s1t[3, 0])

    h = nf_ref[0]

    A1 = jnp.dot(h, mWwT_ref[...], preferred_element_type=f32)
    B1 = jnp.dot(h, mWvT_ref[...], preferred_element_type=f32)
    UA = jnp.dot(A1, ulWmT_ref[...], preferred_element_type=f32)
    UB = jnp.dot(B1, ulWmT_ref[...], preferred_element_type=f32)

    cnt = jnp.stack([jnp.sum(o, axis=0) for o in oh], axis=1)
    s0 = jnp.sum(S, axis=0)
    msum1 = (jnp.dot(cnt, tgt, preferred_element_type=f32)
             + jnp.dot(S.T, A1, preferred_element_type=f32)
             + s0[:, None] * B1)

    gi = jnp.dot(msum1, W_ihT_ref[...], preferred_element_type=f32) + b_ih_ref[...]
    gh = jnp.dot(h, W_hhT_ref[...], preferred_element_type=f32) + b_hh_ref[...]
    r = jax.nn.sigmoid(gi[:, :NF] + gh[:, :NF])
    z = jax.nn.sigmoid(gi[:, NF:2 * NF] + gh[:, NF:2 * NF])
    n = jnp.tanh(gi[:, 2 * NF:] + r * gh[:, 2 * NF:])
    h = (1.0 - z) * n + z * h

    t2g = (oh[0][:, :, None] * T2t[0] + oh[1][:, :, None] * T2t[1]
           + oh[2][:, :, None] * T2t[2] + oh[3][:, :, None] * T2t[3])
    e2 = (t2g + S[:, :, None] * (UA[:, None, :] + UB[None, :, :])
          ).reshape(P, EF)

    x2 = jnp.maximum(jnp.dot(e2, lW1T_ref[...],
                             preferred_element_type=f32) + lb1_ref[...], 0.0)
    sgm2 = jax.nn.sigmoid(jnp.dot(x2, lW2T_ref[...],
                                  preferred_element_type=f32) + lb2_ref[...])
    A2 = jnp.dot(h, mWwT_ref[...], preferred_element_type=f32)
    B2 = jnp.dot(h, mWvT_ref[...], preferred_element_type=f32)
    em2 = jnp.dot(e2, mWeT_ref[...], preferred_element_type=f32)
    MS = em2.shape[1]
    m2 = (sgm2.reshape(N, N, 1)
          * (em2.reshape(N, N, MS) + A2[:, None, :] + B2[None, :, :]
             + mb_ref[...][None, :, :])).reshape(P, MS)
    e3 = (jnp.dot(e2, ulWeT_ref[...], preferred_element_type=f32)
          + jnp.dot(m2, ulWmT_ref[...], preferred_element_type=f32)
          + ulb_ref[...])

    e3T = e3.reshape(N, N, EF).transpose(1, 0, 2).reshape(P, EF)
    rx = jnp.maximum(
        jnp.dot(e3, rW1aT_ref[...], preferred_element_type=f32)
        + jnp.dot(e3T, rW1bT_ref[...], preferred_element_type=f32)
        + rb1_ref[...], 0.0)
    out_ref[0] = (jnp.dot(rx, rW2T_ref[...], preferred_element_type=f32)
                  + rb2_ref[...])


def kernel(edge_ids, node_features, link_labels, event_nums, emb, lW1, lb1,
           lW2, lb2, mW, mb, ulW, ulb, W_ih, W_hh, b_ih, b_hh, rW1, rb1,
           rW2, rb2):
    B, N, _, _ = edge_ids.shape
    NF = node_features.shape[2]
    EF = emb.shape[1]
    P = N * N

    lW1T = lW1.T
    lW2T = lW2.T
    mWwT = mW[:, :NF].T
    mWvT = mW[:, NF:2 * NF].T
    mWeT = mW[:, 2 * NF:].T
    ulWeT = ulW[:, :EF].T
    ulWmT = ulW[:, EF:].T
    W_ihT = W_ih.T
    W_hhT = W_hh.T
    rW1aT = rW1[:, :EF].T
    rW1bT = rW1[:, EF:].T
    rW2T = jnp.pad(rW2.T, ((0, 0), (0, 6)))
    rb2p = jnp.pad(rb2, (0, 6))

    r2 = lambda a: a.reshape(1, -1)

    full = lambda shape: pl.BlockSpec(shape, lambda b: (0,) * len(shape))
    in_specs = [
        pl.BlockSpec((1, N, N), lambda b: (b, 0, 0)),
        pl.BlockSpec((1, N, NF), lambda b: (b, 0, 0)),
        full(emb.shape), full(lW1T.shape), full((1, lb1.shape[0])),
        full(lW2T.shape), full((1, 1)),
        full(mWwT.shape), full(mWvT.shape), full(mWeT.shape),
        full((1, mb.shape[0])),
        full(ulWeT.shape), full(ulWmT.shape), full((1, ulb.shape[0])),
        full(W_ihT.shape), full(W_hhT.shape),
        full((1, b_ih.shape[0])), full((1, b_hh.shape[0])),
        full(rW1aT.shape), full(rW1bT.shape), full((1, rb1.shape[0])),
        full(rW2T.shape), full((1, rb2p.shape[0])),
    ]

    ro = pl.pallas_call(
        _gpnn_body,
        grid=(B,),
        in_specs=in_specs,
        out_specs=pl.BlockSpec((1, P, 16), lambda b: (b, 0, 0)),
        out_shape=jax.ShapeDtypeStruct((B, P, 16), jnp.float32),
    )(edge_ids.reshape(B, N, N), node_features,
      emb, lW1T, r2(lb1), lW2T, r2(lb2),
      mWwT, mWvT, mWeT, r2(mb),
      ulWeT, ulWmT, r2(ulb),
      W_ihT, W_hhT, r2(b_ih), r2(b_hh),
      rW1aT, rW1bT, r2(rb1), rW2T, r2(rb2p))

    L = N * (N - 1) // 2
    ro3 = ro.reshape(B, N, N, 16)
    tri = jnp.concatenate([ro3[:, i, i + 1:, :] for i in range(N - 1)],
                          axis=1)[..., :10]
    return tri.reshape(B, L, 5, 2).transpose(0, 2, 1, 3)


# speedup vs baseline: 4.7965x; 2.4061x over previous
import jax
import jax.numpy as jnp
import numpy as np
from jax.experimental import pallas as pl


def _gpnn_body(ids_ref, nf_ref,
               emb_ref, lW1T_ref, lb1_ref, lW2T_ref, lb2_ref,
               mWwT_ref, mWvT_ref, mWeT_ref, mb_ref,
               ulWeT_ref, ulWmT_ref, ulb_ref,
               W_ihT_ref, W_hhT_ref, b_ih_ref, b_hh_ref,
               rW1T_ref, rb1_ref, rW2T_ref, rb2_ref,
               out_ref):
    N = ids_ref.shape[1]
    P = N * N
    NF = nf_ref.shape[2]
    EF = emb_ref.shape[1]

    f32 = jnp.float32
    ids = ids_ref[0]

    emb = emb_ref[...]
    x1t = jnp.maximum(jnp.dot(emb, lW1T_ref[...],
                              preferred_element_type=f32) + lb1_ref[...], 0.0)
    s1t = jax.nn.sigmoid(jnp.dot(x1t, lW2T_ref[...],
                                 preferred_element_type=f32) + lb2_ref[...])
    tmt = jnp.dot(emb, mWeT_ref[...], preferred_element_type=f32) + mb_ref[...]
    tgt = s1t * tmt
    T2t = (jnp.dot(emb, ulWeT_ref[...], preferred_element_type=f32)
           + ulb_ref[...]
           + s1t * jnp.dot(tmt, ulWmT_ref[...], preferred_element_type=f32))

    oh = [(ids == k).astype(f32) for k in range(4)]
    S = (oh[0] * s1t[0, 0] + oh[1] * s1t[1, 0]
         + oh[2] * s1t[2, 0] + oh[3] * s1t[3, 0])

    h = nf_ref[0]

    A1 = jnp.dot(h, mWwT_ref[...], preferred_element_type=f32)
    B1 = jnp.dot(h, mWvT_ref[...], preferred_element_type=f32)
    UA = jnp.dot(A1, ulWmT_ref[...], preferred_element_type=f32)
    UB = jnp.dot(B1, ulWmT_ref[...], preferred_element_type=f32)

    cnt = jnp.stack([jnp.sum(o, axis=0) for o in oh], axis=1)
    s0 = jnp.sum(S, axis=0)
    msum1 = (jnp.dot(cnt, tgt, preferred_element_type=f32)
             + jnp.dot(S.T, A1, preferred_element_type=f32)
             + s0[:, None] * B1)

    gi = jnp.dot(msum1, W_ihT_ref[...], preferred_element_type=f32) + b_ih_ref[...]
    gh = jnp.dot(h, W_hhT_ref[...], preferred_element_type=f32) + b_hh_ref[...]
    r = jax.nn.sigmoid(gi[:, :NF] + gh[:, :NF])
    z = jax.nn.sigmoid(gi[:, NF:2 * NF] + gh[:, NF:2 * NF])
    n = jnp.tanh(gi[:, 2 * NF:] + r * gh[:, 2 * NF:])
    h = (1.0 - z) * n + z * h

    t2g = (oh[0][:, :, None] * T2t[0] + oh[1][:, :, None] * T2t[1]
           + oh[2][:, :, None] * T2t[2] + oh[3][:, :, None] * T2t[3])
    e2 = (t2g + S[:, :, None] * (UA[:, None, :] + UB[None, :, :])
          ).reshape(P, EF)

    x2 = jnp.maximum(jnp.dot(e2, lW1T_ref[...],
                             preferred_element_type=f32) + lb1_ref[...], 0.0)
    sgm2 = jax.nn.sigmoid(jnp.dot(x2, lW2T_ref[...],
                                  preferred_element_type=f32) + lb2_ref[...])
    A2 = jnp.dot(h, mWwT_ref[...], preferred_element_type=f32)
    B2 = jnp.dot(h, mWvT_ref[...], preferred_element_type=f32)
    em2 = jnp.dot(e2, mWeT_ref[...], preferred_element_type=f32)
    MS = em2.shape[1]
    m2 = (sgm2.reshape(N, N, 1)
          * (em2.reshape(N, N, MS) + A2[:, None, :] + B2[None, :, :]
             + mb_ref[...][None, :, :])).reshape(P, MS)
    e3 = (jnp.dot(e2, ulWeT_ref[...], preferred_element_type=f32)
          + jnp.dot(m2, ulWmT_ref[...], preferred_element_type=f32)
          + ulb_ref[...])

    e33 = e3.reshape(N, N, EF)
    e3T = e33.transpose(1, 0, 2)
    ecat = jnp.concatenate(
        [jnp.concatenate([e33[i, i + 1:, :], e3T[i, i + 1:, :]], axis=1)
         for i in range(N - 1)], axis=0)
    rx = jnp.maximum(jnp.dot(ecat, rW1T_ref[...],
                             preferred_element_type=f32) + rb1_ref[...], 0.0)
    out_ref[0] = (jnp.dot(rx, rW2T_ref[...], preferred_element_type=f32)
                  + rb2_ref[...])


def kernel(edge_ids, node_features, link_labels, event_nums, emb, lW1, lb1,
           lW2, lb2, mW, mb, ulW, ulb, W_ih, W_hh, b_ih, b_hh, rW1, rb1,
           rW2, rb2):
    B, N, _, _ = edge_ids.shape
    NF = node_features.shape[2]
    EF = emb.shape[1]
    P = N * N

    lW1T = lW1.T
    lW2T = lW2.T
    mWwT = mW[:, :NF].T
    mWvT = mW[:, NF:2 * NF].T
    mWeT = mW[:, 2 * NF:].T
    ulWeT = ulW[:, :EF].T
    ulWmT = ulW[:, EF:].T
    W_ihT = W_ih.T
    W_hhT = W_hh.T
    rW1T = rW1.T
    rW2T = jnp.pad(rW2.T, ((0, 0), (0, 6)))
    rb2p = jnp.pad(rb2, (0, 6))

    r2 = lambda a: a.reshape(1, -1)

    full = lambda shape: pl.BlockSpec(shape, lambda b: (0,) * len(shape))
    in_specs = [
        pl.BlockSpec((1, N, N), lambda b: (b, 0, 0)),
        pl.BlockSpec((1, N, NF), lambda b: (b, 0, 0)),
        full(emb.shape), full(lW1T.shape), full((1, lb1.shape[0])),
        full(lW2T.shape), full((1, 1)),
        full(mWwT.shape), full(mWvT.shape), full(mWeT.shape),
        full((1, mb.shape[0])),
        full(ulWeT.shape), full(ulWmT.shape), full((1, ulb.shape[0])),
        full(W_ihT.shape), full(W_hhT.shape),
        full((1, b_ih.shape[0])), full((1, b_hh.shape[0])),
        full(rW1T.shape), full((1, rb1.shape[0])),
        full(rW2T.shape), full((1, rb2p.shape[0])),
    ]
    L = N * (N - 1) // 2

    ro = pl.pallas_call(
        _gpnn_body,
        grid=(B,),
        in_specs=in_specs,
        out_specs=pl.BlockSpec((1, L, 16), lambda b: (b, 0, 0)),
        out_shape=jax.ShapeDtypeStruct((B, L, 16), jnp.float32),
    )(edge_ids.reshape(B, N, N), node_features,
      emb, lW1T, r2(lb1), lW2T, r2(lb2),
      mWwT, mWvT, mWeT, r2(mb),
      ulWeT, ulWmT, r2(ulb),
      W_ihT, W_hhT, r2(b_ih), r2(b_hh),
      rW1T, r2(rb1), rW2T, r2(rb2p))

    tri = ro[..., :10]
    return tri.reshape(B, L, 5, 2).transpose(0, 2, 1, 3)


# speedup vs baseline: 5.9656x; 1.2437x over previous
import jax
import jax.numpy as jnp
import numpy as np
from jax.experimental import pallas as pl


def _gpnn_body(ids_ref, nf_ref,
               emb_ref, lW1T_ref, lb1_ref, lW2T_ref, lb2_ref,
               mWwT_ref, mWvT_ref, mWeT_ref, mb_ref,
               ulWeT_ref, ulWmT_ref, ulb_ref,
               W_ihT_ref, W_hhT_ref, b_ih_ref, b_hh_ref,
               rW1T_ref, rb1_ref, rW2T_ref, rb2_ref,
               out_ref):
    N = ids_ref.shape[1]
    P = N * N
    NF = nf_ref.shape[2]
    EF = emb_ref.shape[1]

    f32 = jnp.float32
    ids = ids_ref[0]

    emb = emb_ref[...]
    x1t = jnp.maximum(jnp.dot(emb, lW1T_ref[...],
                              preferred_element_type=f32) + lb1_ref[...], 0.0)
    s1t = jax.nn.sigmoid(jnp.dot(x1t, lW2T_ref[...],
                                 preferred_element_type=f32) + lb2_ref[...])
    tmt = jnp.dot(emb, mWeT_ref[...], preferred_element_type=f32) + mb_ref[...]
    tgt = s1t * tmt
    T2t = (jnp.dot(emb, ulWeT_ref[...], preferred_element_type=f32)
           + ulb_ref[...]
           + s1t * jnp.dot(tmt, ulWmT_ref[...], preferred_element_type=f32))

    oh = [(ids == k).astype(f32) for k in range(4)]
    S = (oh[0] * s1t[0, 0] + oh[1] * s1t[1, 0]
         + oh[2] * s1t[2, 0] + oh[3] * s1t[3, 0])
    iota4 = jax.lax.broadcasted_iota(jnp.int32, (1, 1, 4), 2)
    OgP = (ids[:, :, None] == iota4).astype(f32).reshape(P, 4)

    h = nf_ref[0]

    A1 = jnp.dot(h, mWwT_ref[...], preferred_element_type=f32)
    B1 = jnp.dot(h, mWvT_ref[...], preferred_element_type=f32)
    UA = jnp.dot(A1, ulWmT_ref[...], preferred_element_type=f32)
    UB = jnp.dot(B1, ulWmT_ref[...], preferred_element_type=f32)

    cnt = jnp.stack([jnp.sum(o, axis=0) for o in oh], axis=1)
    del oh
    s0 = jnp.sum(S, axis=0)
    msum1 = (jnp.dot(cnt, tgt, preferred_element_type=f32)
             + jnp.dot(S.T, A1, preferred_element_type=f32)
             + s0[:, None] * B1)

    gi = jnp.dot(msum1, W_ihT_ref[...], preferred_element_type=f32) + b_ih_ref[...]
    gh = jnp.dot(h, W_hhT_ref[...], preferred_element_type=f32) + b_hh_ref[...]
    r = jax.nn.sigmoid(gi[:, :NF] + gh[:, :NF])
    z = jax.nn.sigmoid(gi[:, NF:2 * NF] + gh[:, NF:2 * NF])
    n = jnp.tanh(gi[:, 2 * NF:] + r * gh[:, 2 * NF:])
    h = (1.0 - z) * n + z * h

    t2g = jnp.dot(OgP, T2t, preferred_element_type=f32)
    Sp = jnp.dot(OgP, s1t, preferred_element_type=f32)
    addUV = (UA[:, None, :] + UB[None, :, :]).reshape(P, EF)
    e2 = t2g + Sp * addUV

    x2 = jnp.maximum(jnp.dot(e2, lW1T_ref[...],
                             preferred_element_type=f32) + lb1_ref[...], 0.0)
    sgm2 = jax.nn.sigmoid(jnp.dot(x2, lW2T_ref[...],
                                  preferred_element_type=f32) + lb2_ref[...])
    A2 = jnp.dot(h, mWwT_ref[...], preferred_element_type=f32)
    B2 = jnp.dot(h, mWvT_ref[...], preferred_element_type=f32)
    em2 = jnp.dot(e2, mWeT_ref[...], preferred_element_type=f32)
    MS = em2.shape[1]
    AB2 = (A2[:, None, :] + B2[None, :, :]).reshape(P, MS)
    m2 = sgm2 * (em2 + AB2 + mb_ref[...])
    e3 = (jnp.dot(e2, ulWeT_ref[...], preferred_element_type=f32)
          + jnp.dot(m2, ulWmT_ref[...], preferred_element_type=f32)
          + ulb_ref[...])

    e33 = e3.reshape(N, N, EF)
    e3T = e33.transpose(1, 0, 2)
    ecat = jnp.concatenate(
        [jnp.concatenate([e33[i, i + 1:, :], e3T[i, i + 1:, :]], axis=1)
         for i in range(N - 1)], axis=0)
    rx = jnp.maximum(jnp.dot(ecat, rW1T_ref[...],
                             preferred_element_type=f32) + rb1_ref[...], 0.0)
    out_ref[0] = (jnp.dot(rx, rW2T_ref[...], preferred_element_type=f32)
                  + rb2_ref[...])


def kernel(edge_ids, node_features, link_labels, event_nums, emb, lW1, lb1,
           lW2, lb2, mW, mb, ulW, ulb, W_ih, W_hh, b_ih, b_hh, rW1, rb1,
           rW2, rb2):
    B, N, _, _ = edge_ids.shape
    NF = node_features.shape[2]
    EF = emb.shape[1]
    P = N * N

    lW1T = lW1.T
    lW2T = lW2.T
    mWwT = mW[:, :NF].T
    mWvT = mW[:, NF:2 * NF].T
    mWeT = mW[:, 2 * NF:].T
    ulWeT = ulW[:, :EF].T
    ulWmT = ulW[:, EF:].T
    W_ihT = W_ih.T
    W_hhT = W_hh.T
    rW1T = rW1.T
    rW2T = jnp.pad(rW2.T, ((0, 0), (0, 6)))
    rb2p = jnp.pad(rb2, (0, 6))

    r2 = lambda a: a.reshape(1, -1)

    full = lambda shape: pl.BlockSpec(shape, lambda b: (0,) * len(shape))
    in_specs = [
        pl.BlockSpec((1, N, N), lambda b: (b, 0, 0)),
        pl.BlockSpec((1, N, NF), lambda b: (b, 0, 0)),
        full(emb.shape), full(lW1T.shape), full((1, lb1.shape[0])),
        full(lW2T.shape), full((1, 1)),
        full(mWwT.shape), full(mWvT.shape), full(mWeT.shape),
        full((1, mb.shape[0])),
        full(ulWeT.shape), full(ulWmT.shape), full((1, ulb.shape[0])),
        full(W_ihT.shape), full(W_hhT.shape),
        full((1, b_ih.shape[0])), full((1, b_hh.shape[0])),
        full(rW1T.shape), full((1, rb1.shape[0])),
        full(rW2T.shape), full((1, rb2p.shape[0])),
    ]
    L = N * (N - 1) // 2

    ro = pl.pallas_call(
        _gpnn_body,
        grid=(B,),
        in_specs=in_specs,
        out_specs=pl.BlockSpec((1, L, 16), lambda b: (b, 0, 0)),
        out_shape=jax.ShapeDtypeStruct((B, L, 16), jnp.float32),
    )(edge_ids.reshape(B, N, N), node_features,
      emb, lW1T, r2(lb1), lW2T, r2(lb2),
      mWwT, mWvT, mWeT, r2(mb),
      ulWeT, ulWmT, r2(ulb),
      W_ihT, W_hhT, r2(b_ih), r2(b_hh),
      rW1T, r2(rb1), rW2T, r2(rb2p))

    tri = ro[..., :10]
    return tri.reshape(B, L, 5, 2).transpose(0, 2, 1, 3)


# speedup vs baseline: 6.3299x; 1.0611x over previous
import jax
import jax.numpy as jnp
import numpy as np
from jax.experimental import pallas as pl


def _gpnn_body(ids_ref, nf_ref,
               emb_ref, lW1T_ref, lb1_ref, lW2rT_ref, lb2_ref,
               mWwT_ref, mWvT_ref, mWeT_ref, mb_ref,
               ulWeT_ref, ulWmT_ref, ulb_ref,
               W_ihT_ref, W_hhT_ref, b_ih_ref, b_hh_ref,
               rW1T_ref, rb1_ref, rW2T_ref, rb2_ref,
               out_ref):
    N = ids_ref.shape[1]
    P = N * N
    NF = nf_ref.shape[2]
    EF = emb_ref.shape[1]

    f32 = jnp.float32
    ids = ids_ref[0]

    emb = emb_ref[...]
    x1t = jnp.maximum(jnp.dot(emb, lW1T_ref[...],
                              preferred_element_type=f32) + lb1_ref[...], 0.0)
    s1t = jax.nn.sigmoid(jnp.dot(x1t, lW2rT_ref[...],
                                 preferred_element_type=f32) + lb2_ref[...])
    tmt = jnp.dot(emb, mWeT_ref[...], preferred_element_type=f32) + mb_ref[...]
    tgt = s1t * tmt
    T2t = (jnp.dot(emb, ulWeT_ref[...], preferred_element_type=f32)
           + ulb_ref[...]
           + s1t * jnp.dot(tmt, ulWmT_ref[...], preferred_element_type=f32))

    oh = [(ids == k).astype(f32) for k in range(4)]
    S = (oh[0] * s1t[0, 0] + oh[1] * s1t[1, 0]
         + oh[2] * s1t[2, 0] + oh[3] * s1t[3, 0])
    iota4 = jax.lax.broadcasted_iota(jnp.int32, (1, 1, 4), 2)
    OgP = (ids[:, :, None] == iota4).astype(f32).reshape(P, 4)

    h = nf_ref[0]

    A1 = jnp.dot(h, mWwT_ref[...], preferred_element_type=f32)
    B1 = jnp.dot(h, mWvT_ref[...], preferred_element_type=f32)
    UA = jnp.dot(A1, ulWmT_ref[...], preferred_element_type=f32)
    UB = jnp.dot(B1, ulWmT_ref[...], preferred_element_type=f32)

    cnt = jnp.stack([jnp.sum(o, axis=0) for o in oh], axis=1)
    del oh
    s0 = jnp.sum(S, axis=0)
    msum1 = (jnp.dot(cnt, tgt, preferred_element_type=f32)
             + jnp.dot(S.T, A1, preferred_element_type=f32)
             + s0[:, None] * B1)

    gi = jnp.dot(msum1, W_ihT_ref[...], preferred_element_type=f32) + b_ih_ref[...]
    gh = jnp.dot(h, W_hhT_ref[...], preferred_element_type=f32) + b_hh_ref[...]
    r = jax.nn.sigmoid(gi[:, :NF] + gh[:, :NF])
    z = jax.nn.sigmoid(gi[:, NF:2 * NF] + gh[:, NF:2 * NF])
    n = jnp.tanh(gi[:, 2 * NF:] + r * gh[:, 2 * NF:])
    h = (1.0 - z) * n + z * h

    t2g = jnp.dot(OgP, T2t, preferred_element_type=f32)
    SpE = jnp.dot(OgP, s1t, preferred_element_type=f32)
    addUV = (UA[:, None, :] + UB[None, :, :]).reshape(P, EF)
    e2 = t2g + SpE * addUV

    x2 = jnp.maximum(jnp.dot(e2, lW1T_ref[...],
                             preferred_element_type=f32) + lb1_ref[...], 0.0)
    sgm2 = jax.nn.sigmoid(jnp.dot(x2, lW2rT_ref[...],
                                  preferred_element_type=f32) + lb2_ref[...])
    A2 = jnp.dot(h, mWwT_ref[...], preferred_element_type=f32)
    B2 = jnp.dot(h, mWvT_ref[...], preferred_element_type=f32)
    em2 = jnp.dot(e2, mWeT_ref[...], preferred_element_type=f32)
    MS = em2.shape[1]
    AB2 = (A2[:, None, :] + B2[None, :, :]).reshape(P, MS)
    m2 = sgm2 * (em2 + AB2 + mb_ref[...])
    e3 = (jnp.dot(e2, ulWeT_ref[...], preferred_element_type=f32)
          + jnp.dot(m2, ulWmT_ref[...], preferred_element_type=f32)
          + ulb_ref[...])

    e33 = e3.reshape(N, N, EF)
    e3T = e33.transpose(1, 0, 2)
    ecat = jnp.concatenate(
        [jnp.concatenate([e33[i, i + 1:, :], e3T[i, i + 1:, :]], axis=1)
         for i in range(N - 1)], axis=0)
    rx = jnp.maximum(jnp.dot(ecat, rW1T_ref[...],
                             preferred_element_type=f32) + rb1_ref[...], 0.0)
    out_ref[0] = (jnp.dot(rx, rW2T_ref[...], preferred_element_type=f32)
                  + rb2_ref[...])


def kernel(edge_ids, node_features, link_labels, event_nums, emb, lW1, lb1,
           lW2, lb2, mW, mb, ulW, ulb, W_ih, W_hh, b_ih, b_hh, rW1, rb1,
           rW2, rb2):
    B, N, _, _ = edge_ids.shape
    NF = node_features.shape[2]
    EF = emb.shape[1]
    P = N * N

    MS = mW.shape[0]
    lW1T = lW1.T
    lW2rT = jnp.tile(lW2.T, (1, MS))
    mWwT = mW[:, :NF].T
    mWvT = mW[:, NF:2 * NF].T
    mWeT = mW[:, 2 * NF:].T
    ulWeT = ulW[:, :EF].T
    ulWmT = ulW[:, EF:].T
    W_ihT = W_ih.T
    W_hhT = W_hh.T
    rW1T = rW1.T
    rW2T = jnp.pad(rW2.T, ((0, 0), (0, 6)))
    rb2p = jnp.pad(rb2, (0, 6))

    r2 = lambda a: a.reshape(1, -1)

    full = lambda shape: pl.BlockSpec(shape, lambda b: (0,) * len(shape))
    in_specs = [
        pl.BlockSpec((1, N, N), lambda b: (b, 0, 0)),
        pl.BlockSpec((1, N, NF), lambda b: (b, 0, 0)),
        full(emb.shape), full(lW1T.shape), full((1, lb1.shape[0])),
        full(lW2rT.shape), full((1, 1)),
        full(mWwT.shape), full(mWvT.shape), full(mWeT.shape),
        full((1, mb.shape[0])),
        full(ulWeT.shape), full(ulWmT.shape), full((1, ulb.shape[0])),
        full(W_ihT.shape), full(W_hhT.shape),
        full((1, b_ih.shape[0])), full((1, b_hh.shape[0])),
        full(rW1T.shape), full((1, rb1.shape[0])),
        full(rW2T.shape), full((1, rb2p.shape[0])),
    ]
    L = N * (N - 1) // 2

    ro = pl.pallas_call(
        _gpnn_body,
        grid=(B,),
        in_specs=in_specs,
        out_specs=pl.BlockSpec((1, L, 16), lambda b: (b, 0, 0)),
        out_shape=jax.ShapeDtypeStruct((B, L, 16), jnp.float32),
    )(edge_ids.reshape(B, N, N), node_features,
      emb, lW1T, r2(lb1), lW2rT, r2(lb2),
      mWwT, mWvT, mWeT, r2(mb),
      ulWeT, ulWmT, r2(ulb),
      W_ihT, W_hhT, r2(b_ih), r2(b_hh),
      rW1T, r2(rb1), rW2T, r2(rb2p))

    tri = ro[..., :10]
    return tri.reshape(B, L, 5, 2).transpose(0, 2, 1, 3)


# speedup vs baseline: 6.9006x; 1.0902x over previous
import jax
import jax.numpy as jnp
from jax.experimental import pallas as pl


def _dotT(x, w):
    return jax.lax.dot_general(x, w, (((1,), (1,)), ((), ())),
                               preferred_element_type=jnp.float32)


def _gpnn_body(ids_ref, nf_ref,
               emb_ref, lW1_ref, lb1_ref, lW2_ref, lb2_ref,
               mW_ref, mb_ref, ulW_ref, ulb_ref,
               W_ih_ref, W_hh_ref, b_ih_ref, b_hh_ref,
               rW1_ref, rb1_ref, rW2_ref, rb2_ref,
               out_ref):
    N = ids_ref.shape[1]
    P = N * N
    NF = nf_ref.shape[2]
    EF = emb_ref.shape[1]

    f32 = jnp.float32
    ids = ids_ref[0]

    lW1 = lW1_ref[...]
    mW = mW_ref[...]
    mWw, mWv, mWe = mW[:, :NF], mW[:, NF:2 * NF], mW[:, 2 * NF:]
    ulW = ulW_ref[...]
    ulWe, ulWm = ulW[:, :EF], ulW[:, EF:]
    MS = mW.shape[0]
    lW2r = jnp.broadcast_to(lW2_ref[...], (MS, lW2_ref.shape[1]))

    emb = emb_ref[...]
    x1t = jnp.maximum(_dotT(emb, lW1) + lb1_ref[...], 0.0)
    s1t = jax.nn.sigmoid(_dotT(x1t, lW2r) + lb2_ref[...])
    tmt = _dotT(emb, mWe) + mb_ref[...]
    tgt = s1t * tmt
    T2t = _dotT(emb, ulWe) + ulb_ref[...] + s1t * _dotT(tmt, ulWm)

    oh = [(ids == k).astype(f32) for k in range(4)]
    S = (oh[0] * s1t[0, 0] + oh[1] * s1t[1, 0]
         + oh[2] * s1t[2, 0] + oh[3] * s1t[3, 0])
    iota4 = jax.lax.broadcasted_iota(jnp.int32, (1, 1, 4), 2)
    OgP = (ids[:, :, None] == iota4).astype(f32).reshape(P, 4)

    h = nf_ref[0]

    A1 = _dotT(h, mWw)
    B1 = _dotT(h, mWv)
    UA = _dotT(A1, ulWm)
    UB = _dotT(B1, ulWm)

    cnt = jnp.stack([jnp.sum(o, axis=0) for o in oh], axis=1)
    del oh
    s0 = jnp.sum(S, axis=0)
    msum1 = (jnp.dot(cnt, tgt, preferred_element_type=f32)
             + jnp.dot(S.T, A1, preferred_element_type=f32)
             + s0[:, None] * B1)

    gi = _dotT(msum1, W_ih_ref[...]) + b_ih_ref[...]
    gh = _dotT(h, W_hh_ref[...]) + b_hh_ref[...]
    r = jax.nn.sigmoid(gi[:, :NF] + gh[:, :NF])
    z = jax.nn.sigmoid(gi[:, NF:2 * NF] + gh[:, NF:2 * NF])
    n = jnp.tanh(gi[:, 2 * NF:] + r * gh[:, 2 * NF:])
    h = (1.0 - z) * n + z * h

    t2g = jnp.dot(OgP, T2t, preferred_element_type=f32)
    SpE = jnp.dot(OgP, s1t, preferred_element_type=f32)
    addUV = (UA[:, None, :] + UB[None, :, :]).reshape(P, EF)
    e2 = t2g + SpE * addUV

    x2 = jnp.maximum(_dotT(e2, lW1) + lb1_ref[...], 0.0)
    sgm2 = jax.nn.sigmoid(_dotT(x2, lW2r) + lb2_ref[...])
    A2 = _dotT(h, mWw)
    B2 = _dotT(h, mWv)
    em2 = _dotT(e2, mWe)
    AB2 = (A2[:, None, :] + B2[None, :, :]).reshape(P, MS)
    m2 = sgm2 * (em2 + AB2 + mb_ref[...])
    e3 = _dotT(e2, ulWe) + _dotT(m2, ulWm) + ulb_ref[...]

    e33 = e3.reshape(N, N, EF)
    e3T = e33.transpose(1, 0, 2)
    ecat = jnp.concatenate(
        [jnp.concatenate([e33[i, i + 1:, :], e3T[i, i + 1:, :]], axis=1)
         for i in range(N - 1)], axis=0)
    rx = jnp.maximum(_dotT(ecat, rW1_ref[...]) + rb1_ref[...], 0.0)
    out_ref[0, :, :10] = _dotT(rx, rW2_ref[...]) + rb2_ref[...]


def kernel(edge_ids, node_features, link_labels, event_nums, emb, lW1, lb1,
           lW2, lb2, mW, mb, ulW, ulb, W_ih, W_hh, b_ih, b_hh, rW1, rb1,
           rW2, rb2):
    B, N, _, _ = edge_ids.shape
    NF = node_features.shape[2]
    L = N * (N - 1) // 2

    r2 = lambda a: a.reshape(1, -1)

    full = lambda shape: pl.BlockSpec(shape, lambda b: (0,) * len(shape))
    in_specs = [
        pl.BlockSpec((1, N, N), lambda b: (b, 0, 0)),
        pl.BlockSpec((1, N, NF), lambda b: (b, 0, 0)),
        full(emb.shape), full(lW1.shape), full((1, lb1.shape[0])),
        full(lW2.shape), full((1, 1)),
        full(mW.shape), full((1, mb.shape[0])),
        full(ulW.shape), full((1, ulb.shape[0])),
        full(W_ih.shape), full(W_hh.shape),
        full((1, b_ih.shape[0])), full((1, b_hh.shape[0])),
        full(rW1.shape), full((1, rb1.shape[0])),
        full(rW2.shape), full((1, rb2.shape[0])),
    ]

    ro = pl.pallas_call(
        _gpnn_body,
        grid=(B,),
        in_specs=in_specs,
        out_specs=pl.BlockSpec((1, L, 16), lambda b: (b, 0, 0)),
        out_shape=jax.ShapeDtypeStruct((B, L, 16), jnp.float32),
    )(edge_ids.reshape(B, N, N), node_features,
      emb, lW1, r2(lb1), lW2, r2(lb2),
      mW, r2(mb), ulW, r2(ulb),
      W_ih, W_hh, r2(b_ih), r2(b_hh),
      rW1, r2(rb1), rW2, r2(rb2))

    tri = ro[..., :10]
    return tri.reshape(B, L, 5, 2).transpose(0, 2, 1, 3)
